# baseline (speedup 1.0000x reference)
import functools

import jax
import jax.numpy as jnp
import numpy as np
from jax import lax
from jax.experimental import pallas as pl
from jax.experimental.pallas import tpu as pltpu
from jax.experimental.pallas import tpu_sc as plsc

_ANCHORS = ((1.19, 1.98), (2.79, 4.59), (4.53, 8.92), (8.06, 5.29),
            (10.32, 10.65))
_A = 5
_STRIDE = 85
_NC = 80
_HW = 1024
_Q = 5120
_K = 300
_S = 384
_NMS_THRESH = 0.45
_DS = 32.0
_NV = _Q // 16


def _split3(a):
    b1 = a.astype(jnp.bfloat16)
    r1 = a - b1.astype(jnp.float32)
    b2 = r1.astype(jnp.bfloat16)
    b3 = (r1 - b2.astype(jnp.float32)).astype(jnp.bfloat16)
    return jnp.concatenate([b1, b2, b3], axis=0)


def _dott(h, p):
    n = p.shape[0]
    o = lax.dot_general(h.astype(jnp.bfloat16), _split3(p),
                        (((1,), (1,)), ((), ())),
                        preferred_element_type=jnp.float32)
    return o[:, :n] + o[:, n:2 * n] + o[:, 2 * n:3 * n]


def _decode_kernel(x_ref, sf_ref, sz_ref, pay_ref):
    h_img = sz_ref[0, 0, 0]
    w_img = sz_ref[0, 0, 1]
    lane1 = lax.broadcasted_iota(jnp.int32, (_HW,), 0)
    fx = jnp.bitwise_and(lane1, 31).astype(jnp.float32)
    fy = lax.shift_right_logical(lane1, 5).astype(jnp.float32)

    x1s, y1s, x2s, y2s, lbls = [], [], [], [], []
    for a in range(_A):
        base = a * _STRIDE
        logits = x_ref[0, base:base + _NC, :]
        m = jnp.max(logits, axis=0)
        r80 = lax.broadcasted_iota(jnp.int32, (_NC, _HW), 0).astype(jnp.float32)
        lbl = jnp.min(jnp.where(logits == m[None, :], r80, 1e9), axis=0)
        lbls.append(lbl[None, :])
        zx = x_ref[0, base + _NC, :]
        zy = x_ref[0, base + _NC + 1, :]
        zw = x_ref[0, base + _NC + 2, :]
        zh = x_ref[0, base + _NC + 3, :]
        sigx = 1.0 / (1.0 + jnp.exp(-zx))
        sigy = 1.0 / (1.0 + jnp.exp(-zy))
        cx = (fx + sigx) * _DS
        cy = (fy + sigy) * _DS
        aw, ah = _ANCHORS[a]
        bw = aw * jnp.exp(zw) * _DS
        bh = ah * jnp.exp(zh) * _DS
        x1s.append(jnp.clip(cx - bw / 2.0, 0.0, w_img)[None, :])
        x2s.append(jnp.clip(cx + bw / 2.0, 0.0, w_img)[None, :])
        y1s.append(jnp.clip(cy - bh / 2.0, 0.0, h_img)[None, :])
        y2s.append(jnp.clip(cy + bh / 2.0, 0.0, h_img)[None, :])

    rows = [jnp.concatenate(x1s, axis=1), jnp.concatenate(y1s, axis=1),
            jnp.concatenate(x2s, axis=1), jnp.concatenate(y2s, axis=1),
            sf_ref[0].reshape(1, _Q), jnp.concatenate(lbls, axis=1)]
    pay_ref[0] = jnp.concatenate(rows, axis=0)


def _sc_select_body(sf_hbm, pay_hbm, out_hbm, sbuf, pbuf, obuf, scomp, ccomp,
                    tbuf):
    wid = lax.axis_index("s") * 2 + lax.axis_index("c")
    pltpu.sync_copy(sf_hbm.at[wid], sbuf)
    pltpu.sync_copy(pay_hbm.at[wid], pbuf)

    iota16 = lax.broadcasted_iota(jnp.int32, (16,), 0)
    zero16 = jnp.zeros((16,), jnp.float32)

    def splat_total(x):
        for d in (1, 2, 4, 8):
            tbuf[pl.ds(0, 16)] = x
            x = x + plsc.load_gather(tbuf, [jnp.bitwise_xor(iota16, d)])
        return x

    def mask_count(m):
        return splat_total(jnp.where(m, 1, 0))

    def zb(i, _):
        obuf[pl.ds(i * 16, 16)] = zero16
        return 0

    lax.fori_loop(0, 8 * _S // 16, zb, 0)

    zero16i = jnp.zeros((16,), jnp.int32)

    def count_ge(t):
        def body(i, acc):
            x = sbuf[pl.ds(i * 16, 16)]
            return acc + jnp.where(x >= t, 1, 0)

        return splat_total(lax.fori_loop(0, _NV, body, zero16i, unroll=8))

    def bs1(_, carry):
        lo, hi = carry
        mid = (lo + hi) * 0.5
        ok = count_ge(mid) >= 300
        return jnp.where(ok, mid, lo), jnp.where(ok, hi, mid)

    v, _ = lax.fori_loop(0, 27, bs1,
                         (jnp.zeros((16,), jnp.float32),
                          jnp.ones((16,), jnp.float32)))

    def count_gt(t):
        def body(i, acc):
            x = sbuf[pl.ds(i * 16, 16)]
            return acc + jnp.where(x > t, 1, 0)

        return splat_total(lax.fori_loop(0, _NV, body, zero16i, unroll=8))

    need = 300 - count_gt(v)

    def count_tie_le(c):
        def body(i, acc):
            q = iota16 + i * 16
            cand = (jnp.bitwise_and(q, 1023) * 5
                    + lax.shift_right_logical(q, 10))
            x = sbuf[pl.ds(i * 16, 16)]
            return acc + jnp.where((x == v) & (cand <= c), 1, 0)

        return splat_total(lax.fori_loop(0, _NV, body, zero16i, unroll=8))

    def bs2(_, carry):
        lo, hi = carry
        mid = lax.shift_right_arithmetic(lo + hi, 1)
        ok = count_tie_le(mid) >= need
        return jnp.where(ok, lo, mid), jnp.where(ok, mid, hi)

    n_tie = count_tie_le(jnp.full((16,), 5119, jnp.int32))

    def bisect_cut(_):
        _, c = lax.fori_loop(0, 14, bs2,
                             (jnp.full((16,), -1, jnp.int32),
                              jnp.full((16,), 5119, jnp.int32)))
        return c

    cut = lax.cond(n_tie[0] <= need[0],
                   lambda _: jnp.full((16,), 5119, jnp.int32),
                   bisect_cut, 0)

    def comp_body(i, off):
        q = iota16 + i * 16
        cand = (jnp.bitwise_and(q, 1023) * 5
                + lax.shift_right_logical(q, 10))
        x = sbuf[pl.ds(i * 16, 16)]
        m = (x > v) | ((x == v) & (cand <= cut))
        plsc.store_compressed(scomp.at[pl.ds(off, 16)], x, mask=m)
        plsc.store_compressed(ccomp.at[pl.ds(off, 16)],
                              cand.astype(jnp.float32), mask=m)
        return off + mask_count(m)[0]

    def pad_body(i, _):
        scomp[pl.ds(i * 16, 16)] = zero16
        ccomp[pl.ds(i * 16, 16)] = jnp.full((16,), 8191.0, jnp.float32)
        return 0

    lax.fori_loop(0, _S // 16, pad_body, 0)
    lax.fori_loop(0, _NV, comp_body, 0)

    def rank_body(i, _):
        s_i = scomp[pl.ds(i * 16, 16)]
        c_i = ccomp[pl.ds(i * 16, 16)]

        def inner(j, acc):
            s_jv = scomp[pl.ds(j * 16, 16)]
            c_jv = ccomp[pl.ds(j * 16, 16)]
            for l in range(16):
                sj = s_jv[l]
                cj = c_jv[l]
                above = (sj > s_i) | ((sj == s_i) & (cj < c_i))
                acc = acc + jnp.where(above, 1.0, 0.0)
            return acc

        rank = lax.fori_loop(0, 19, inner, zero16, unroll=4)
        ranki = rank.astype(jnp.int32)
        candi = c_i.astype(jnp.int32)
        qv = (jnp.bitwise_and(candi, jnp.int32(0x7FFF)) % 5) * 1024 + candi // 5
        valid = ranki < _S
        ridx = jnp.where(valid, ranki, _S - 1)
        for r in range(6):
            vals = plsc.load_gather(pbuf, [qv + r * _Q])
            plsc.store_scatter(obuf, [ridx + r * _S], vals, mask=valid)
        return 0

    lax.fori_loop(0, 19, rank_body, 0)
    pltpu.sync_copy(obuf, out_hbm.at[wid])


def _sc_select(sf2, pay):
    mesh = plsc.VectorSubcoreMesh(core_axis_name="c", subcore_axis_name="s")
    return pl.kernel(
        _sc_select_body,
        mesh=mesh,
        compiler_params=pltpu.CompilerParams(needs_layout_passes=False),
        out_type=jax.ShapeDtypeStruct((32, 8 * _S), jnp.float32),
        scratch_types=[
            pltpu.VMEM((_Q,), jnp.float32),
            pltpu.VMEM((6 * _Q,), jnp.float32),
            pltpu.VMEM((8 * _S,), jnp.float32),
            pltpu.VMEM((_S,), jnp.float32),
            pltpu.VMEM((_S,), jnp.float32),
            pltpu.VMEM((128,), jnp.int32),
        ],
    )(sf2, pay)


def _nms_kernel(comp_ref, out_ref):
    acc = comp_ref[0]
    riota = lax.broadcasted_iota(jnp.int32, (_S, _S), 0).astype(jnp.float32)
    ciota = lax.broadcasted_iota(jnp.int32, (_S, _S), 1).astype(jnp.float32)
    ident = jnp.where(riota == ciota, 1.0, 0.0)
    score_r = acc[4][None, :]
    lb = acc[5][None, :]
    off = lb * 4096.0
    bx1 = acc[0][None, :] + off
    by1 = acc[1][None, :] + off
    bx2 = acc[2][None, :] + off
    by2 = acc[3][None, :] + off
    colsB = _dott(ident, jnp.concatenate([bx1, by1, bx2, by2], axis=0))
    bx1c = colsB[:, 0:1]
    by1c = colsB[:, 1:2]
    bx2c = colsB[:, 2:3]
    by2c = colsB[:, 3:4]
    area_r = jnp.maximum(bx2 - bx1, 0.0) * jnp.maximum(by2 - by1, 0.0)
    area_c = jnp.maximum(bx2c - bx1c, 0.0) * jnp.maximum(by2c - by1c, 0.0)
    ix1 = jnp.maximum(bx1c, bx1)
    iy1 = jnp.maximum(by1c, by1)
    ix2 = jnp.minimum(bx2c, bx2)
    iy2 = jnp.minimum(by2c, by2)
    inter = jnp.maximum(ix2 - ix1, 0.0) * jnp.maximum(iy2 - iy1, 0.0)
    iou = inter / (area_c + area_r - inter + 1e-9)
    supp = jnp.max(jnp.where(riota < ciota, iou, 0.0), axis=0, keepdims=True)
    keep = (supp <= _NMS_THRESH) & (score_r > 0.0)
    keepf = keep.astype(jnp.float32)
    out_ref[0] = jnp.concatenate([
        acc[0][None, :] * keepf, acc[1][None, :] * keepf,
        acc[2][None, :] * keepf, acc[3][None, :] * keepf,
        acc[4][None, :] * keepf,
        jnp.where(keep, lb, -1.0),
        jnp.zeros((2, _S), jnp.float32),
    ], axis=0)


@jax.jit
def kernel(boxes_offset, image_sizes):
    n, c, hh, ww = boxes_offset.shape
    x = boxes_offset.reshape(n, c, hh * ww)
    obj = x.reshape(n, _A, _STRIDE, hh * ww)[:, :, _NC + 4, :]
    sc = jax.nn.sigmoid(obj)
    sf = jnp.where(sc > 0.5, sc, 0.0)
    sz = image_sizes.astype(jnp.float32).reshape(n, 1, 2)

    pay = pl.pallas_call(
        _decode_kernel,
        grid=(n,),
        in_specs=[
            pl.BlockSpec((1, c, hh * ww), lambda i: (i, 0, 0)),
            pl.BlockSpec((1, _A, hh * ww), lambda i: (i, 0, 0)),
            pl.BlockSpec((1, 1, 2), lambda i: (i, 0, 0),
                         memory_space=pltpu.SMEM),
        ],
        out_specs=pl.BlockSpec((1, 6, _Q), lambda i: (i, 0, 0)),
        out_shape=jax.ShapeDtypeStruct((n, 6, _Q), jnp.float32),
        compiler_params=pltpu.CompilerParams(
            dimension_semantics=("arbitrary",)),
    )(x, sf, sz)

    comp = _sc_select(sf.reshape(n, _Q), pay.reshape(n, 6 * _Q))
    comp = comp.reshape(n, 8, _S)

    out = pl.pallas_call(
        _nms_kernel,
        grid=(n,),
        in_specs=[pl.BlockSpec((1, 8, _S), lambda i: (i, 0, 0))],
        out_specs=pl.BlockSpec((1, 8, _S), lambda i: (i, 0, 0)),
        out_shape=jax.ShapeDtypeStruct((n, 8, _S), jnp.float32),
        compiler_params=pltpu.CompilerParams(
            dimension_semantics=("arbitrary",)),
    )(comp)

    boxes = jnp.transpose(out[:, 0:4, :_K], (0, 2, 1))
    scores = out[:, 4, :_K]
    labels = out[:, 5, :_K].astype(jnp.int32)
    return boxes, scores, labels


# speedup vs baseline: 2.1157x; 2.1157x over previous
import jax
import jax.numpy as jnp
from jax import lax
from jax.experimental import pallas as pl
from jax.experimental.pallas import tpu as pltpu
from jax.experimental.pallas import tpu_sc as plsc

_ANCHORS = ((1.19, 1.98), (2.79, 4.59), (4.53, 8.92), (8.06, 5.29),
            (10.32, 10.65))
_A = 5
_STRIDE = 85
_NC = 80
_HW = 1024
_Q = 5120
_K = 300
_S = 384
_NMS_THRESH = 0.45
_DS = 32.0
_NV = _Q // 16


def _split3(a):
    b1 = a.astype(jnp.bfloat16)
    r1 = a - b1.astype(jnp.float32)
    b2 = r1.astype(jnp.bfloat16)
    b3 = (r1 - b2.astype(jnp.float32)).astype(jnp.bfloat16)
    return jnp.concatenate([b1, b2, b3], axis=0)


def _dott(h, p):
    n = p.shape[0]
    o = lax.dot_general(h.astype(jnp.bfloat16), _split3(p),
                        (((1,), (1,)), ((), ())),
                        preferred_element_type=jnp.float32)
    return o[:, :n] + o[:, n:2 * n] + o[:, 2 * n:3 * n]


def _decode_kernel(x_ref, sf_ref, sz_ref, pay_ref):
    h_img = sz_ref[0, 0, 0]
    w_img = sz_ref[0, 0, 1]
    lane1 = lax.broadcasted_iota(jnp.int32, (_HW,), 0)
    fx = jnp.bitwise_and(lane1, 31).astype(jnp.float32)
    fy = lax.shift_right_logical(lane1, 5).astype(jnp.float32)

    x1s, y1s, x2s, y2s, lbls = [], [], [], [], []
    for a in range(_A):
        base = a * _STRIDE
        logits = x_ref[0, base:base + _NC, :]
        m = jnp.max(logits, axis=0)
        r80 = lax.broadcasted_iota(jnp.int32, (_NC, _HW), 0).astype(jnp.float32)
        lbl = jnp.min(jnp.where(logits == m[None, :], r80, 1e9), axis=0)
        lbls.append(lbl[None, :])
        zx = x_ref[0, base + _NC, :]
        zy = x_ref[0, base + _NC + 1, :]
        zw = x_ref[0, base + _NC + 2, :]
        zh = x_ref[0, base + _NC + 3, :]
        sigx = 1.0 / (1.0 + jnp.exp(-zx))
        sigy = 1.0 / (1.0 + jnp.exp(-zy))
        cx = (fx + sigx) * _DS
        cy = (fy + sigy) * _DS
        aw, ah = _ANCHORS[a]
        bw = aw * jnp.exp(zw) * _DS
        bh = ah * jnp.exp(zh) * _DS
        x1s.append(jnp.clip(cx - bw / 2.0, 0.0, w_img)[None, :])
        x2s.append(jnp.clip(cx + bw / 2.0, 0.0, w_img)[None, :])
        y1s.append(jnp.clip(cy - bh / 2.0, 0.0, h_img)[None, :])
        y2s.append(jnp.clip(cy + bh / 2.0, 0.0, h_img)[None, :])

    rows = [jnp.concatenate(x1s, axis=1), jnp.concatenate(y1s, axis=1),
            jnp.concatenate(x2s, axis=1), jnp.concatenate(y2s, axis=1),
            sf_ref[0].reshape(1, _Q), jnp.concatenate(lbls, axis=1)]
    pay_ref[0] = jnp.concatenate(rows, axis=0)


def _sc_select_body(sf_hbm, pay_hbm, out_hbm, sbuf, pbuf, obuf, scomp, ccomp,
                    tbuf):
    wid = lax.axis_index("s") * 2 + lax.axis_index("c")
    pltpu.sync_copy(sf_hbm.at[wid], sbuf)
    pltpu.sync_copy(pay_hbm.at[wid], pbuf)

    iota16 = lax.broadcasted_iota(jnp.int32, (16,), 0)
    zero16 = jnp.zeros((16,), jnp.float32)

    def splat_total(x):
        for d in (1, 2, 4, 8):
            tbuf[pl.ds(0, 16)] = x
            x = x + plsc.load_gather(tbuf, [jnp.bitwise_xor(iota16, d)])
        return x

    def mask_count(m):
        return splat_total(jnp.where(m, 1, 0))

    def zb(i, _):
        obuf[pl.ds(i * 16, 16)] = zero16
        return 0

    lax.fori_loop(0, 8 * _S // 16, zb, 0)

    zero16i = jnp.zeros((16,), jnp.int32)

    def count_ge(t):
        def body(i, acc):
            x = sbuf[pl.ds(i * 16, 16)]
            return acc + jnp.where(x >= t, 1, 0)

        return splat_total(lax.fori_loop(0, _NV, body, zero16i, unroll=8))

    def bs1(_, carry):
        lo, hi = carry
        mid = (lo + hi) * 0.5
        ok = count_ge(mid) >= 300
        return jnp.where(ok, mid, lo), jnp.where(ok, hi, mid)

    v, _ = lax.fori_loop(0, 27, bs1,
                         (jnp.zeros((16,), jnp.float32),
                          jnp.ones((16,), jnp.float32)))

    def count_gt(t):
        def body(i, acc):
            x = sbuf[pl.ds(i * 16, 16)]
            return acc + jnp.where(x > t, 1, 0)

        return splat_total(lax.fori_loop(0, _NV, body, zero16i, unroll=8))

    need = 300 - count_gt(v)

    def count_tie_le(c):
        def body(i, acc):
            q = iota16 + i * 16
            cand = (jnp.bitwise_and(q, 1023) * 5
                    + lax.shift_right_logical(q, 10))
            x = sbuf[pl.ds(i * 16, 16)]
            return acc + jnp.where((x == v) & (cand <= c), 1, 0)

        return splat_total(lax.fori_loop(0, _NV, body, zero16i, unroll=8))

    def bs2(_, carry):
        lo, hi = carry
        mid = lax.shift_right_arithmetic(lo + hi, 1)
        ok = count_tie_le(mid) >= need
        return jnp.where(ok, lo, mid), jnp.where(ok, mid, hi)

    n_tie = count_tie_le(jnp.full((16,), 5119, jnp.int32))

    def bisect_cut(_):
        _, c = lax.fori_loop(0, 14, bs2,
                             (jnp.full((16,), -1, jnp.int32),
                              jnp.full((16,), 5119, jnp.int32)))
        return c

    cut = lax.cond(n_tie[0] <= need[0],
                   lambda _: jnp.full((16,), 5119, jnp.int32),
                   bisect_cut, 0)

    def comp_body(i, off):
        q = iota16 + i * 16
        cand = (jnp.bitwise_and(q, 1023) * 5
                + lax.shift_right_logical(q, 10))
        x = sbuf[pl.ds(i * 16, 16)]
        m = (x > v) | ((x == v) & (cand <= cut))
        plsc.store_compressed(scomp.at[pl.ds(off, 16)], x, mask=m)
        plsc.store_compressed(ccomp.at[pl.ds(off, 16)],
                              cand.astype(jnp.float32), mask=m)
        return off + mask_count(m)[0]

    def pad_body(i, _):
        scomp[pl.ds(i * 16, 16)] = zero16
        ccomp[pl.ds(i * 16, 16)] = jnp.full((16,), 8191.0, jnp.float32)
        return 0

    lax.fori_loop(0, _S // 16, pad_body, 0)
    lax.fori_loop(0, _NV, comp_body, 0)

    def rank_body(i, _):
        s_i = scomp[pl.ds(i * 16, 16)]
        c_i = ccomp[pl.ds(i * 16, 16)]

        def inner(j, acc):
            s_jv = scomp[pl.ds(j * 16, 16)]
            c_jv = ccomp[pl.ds(j * 16, 16)]
            for l in range(16):
                sj = s_jv[l]
                cj = c_jv[l]
                above = (sj > s_i) | ((sj == s_i) & (cj < c_i))
                acc = acc + jnp.where(above, 1.0, 0.0)
            return acc

        rank = lax.fori_loop(0, 19, inner, zero16, unroll=4)
        ranki = rank.astype(jnp.int32)
        candi = c_i.astype(jnp.int32)
        qv = (jnp.bitwise_and(candi, jnp.int32(0x7FFF)) % 5) * 1024 + candi // 5
        valid = ranki < _S
        ridx = jnp.where(valid, ranki, _S - 1)
        for r in range(6):
            vals = plsc.load_gather(pbuf, [qv + r * _Q])
            plsc.store_scatter(obuf, [ridx + r * _S], vals, mask=valid)
        return 0

    lax.fori_loop(0, 19, rank_body, 0)
    pltpu.sync_copy(obuf, out_hbm.at[wid])


def _sc_select(sf2, pay):
    mesh = plsc.VectorSubcoreMesh(core_axis_name="c", subcore_axis_name="s")
    return pl.kernel(
        _sc_select_body,
        mesh=mesh,
        compiler_params=pltpu.CompilerParams(needs_layout_passes=False),
        out_type=jax.ShapeDtypeStruct((32, 8 * _S), jnp.float32),
        scratch_types=[
            pltpu.VMEM((_Q,), jnp.float32),
            pltpu.VMEM((6 * _Q,), jnp.float32),
            pltpu.VMEM((8 * _S,), jnp.float32),
            pltpu.VMEM((_S,), jnp.float32),
            pltpu.VMEM((_S,), jnp.float32),
            pltpu.VMEM((128,), jnp.int32),
        ],
    )(sf2, pay)


def _nms_kernel(comp_ref, out_ref):
    acc = comp_ref[0]
    riota = lax.broadcasted_iota(jnp.int32, (_S, _S), 0).astype(jnp.float32)
    ciota = lax.broadcasted_iota(jnp.int32, (_S, _S), 1).astype(jnp.float32)
    ident = jnp.where(riota == ciota, 1.0, 0.0)
    score_r = acc[4][None, :]
    lb = acc[5][None, :]
    off = lb * 4096.0
    bx1 = acc[0][None, :] + off
    by1 = acc[1][None, :] + off
    bx2 = acc[2][None, :] + off
    by2 = acc[3][None, :] + off
    colsB = _dott(ident, jnp.concatenate([bx1, by1, bx2, by2], axis=0))
    bx1c = colsB[:, 0:1]
    by1c = colsB[:, 1:2]
    bx2c = colsB[:, 2:3]
    by2c = colsB[:, 3:4]
    area_r = jnp.maximum(bx2 - bx1, 0.0) * jnp.maximum(by2 - by1, 0.0)
    area_c = jnp.maximum(bx2c - bx1c, 0.0) * jnp.maximum(by2c - by1c, 0.0)
    ix1 = jnp.maximum(bx1c, bx1)
    iy1 = jnp.maximum(by1c, by1)
    ix2 = jnp.minimum(bx2c, bx2)
    iy2 = jnp.minimum(by2c, by2)
    inter = jnp.maximum(ix2 - ix1, 0.0) * jnp.maximum(iy2 - iy1, 0.0)
    iou = inter / (area_c + area_r - inter + 1e-9)
    supp = jnp.max(jnp.where(riota < ciota, iou, 0.0), axis=0, keepdims=True)
    keep = (supp <= _NMS_THRESH) & (score_r > 0.0)
    keepf = keep.astype(jnp.float32)
    out_ref[0] = jnp.concatenate([
        acc[0][None, :] * keepf, acc[1][None, :] * keepf,
        acc[2][None, :] * keepf, acc[3][None, :] * keepf,
        acc[4][None, :] * keepf,
        jnp.where(keep, lb, -1.0),
        jnp.zeros((2, _S), jnp.float32),
    ], axis=0)


@jax.jit
def kernel(boxes_offset, image_sizes):
    n, c, hh, ww = boxes_offset.shape
    x = boxes_offset.reshape(n, c, hh * ww)
    obj = x[:, _NC + 4::_STRIDE, :]
    sc = jax.nn.sigmoid(obj)
    sf = jnp.where(sc > 0.5, sc, 0.0)
    sz = image_sizes.astype(jnp.float32).reshape(n, 1, 2)

    pay = pl.pallas_call(
        _decode_kernel,
        grid=(n,),
        in_specs=[
            pl.BlockSpec((1, c, hh * ww), lambda i: (i, 0, 0)),
            pl.BlockSpec((1, _A, hh * ww), lambda i: (i, 0, 0)),
            pl.BlockSpec((1, 1, 2), lambda i: (i, 0, 0),
                         memory_space=pltpu.SMEM),
        ],
        out_specs=pl.BlockSpec((1, 6, _Q), lambda i: (i, 0, 0)),
        out_shape=jax.ShapeDtypeStruct((n, 6, _Q), jnp.float32),
        compiler_params=pltpu.CompilerParams(
            dimension_semantics=("arbitrary",)),
    )(x, sf, sz)

    comp = _sc_select(sf.reshape(n, _Q), pay.reshape(n, 6 * _Q))
    comp = comp.reshape(n, 8, _S)

    out = pl.pallas_call(
        _nms_kernel,
        grid=(n,),
        in_specs=[pl.BlockSpec((1, 8, _S), lambda i: (i, 0, 0))],
        out_specs=pl.BlockSpec((1, 8, _S), lambda i: (i, 0, 0)),
        out_shape=jax.ShapeDtypeStruct((n, 8, _S), jnp.float32),
        compiler_params=pltpu.CompilerParams(
            dimension_semantics=("arbitrary",)),
    )(comp)

    boxes = jnp.transpose(out[:, 0:4, :_K], (0, 2, 1))
    scores = out[:, 4, :_K]
    labels = out[:, 5, :_K].astype(jnp.int32)
    return boxes, scores, labels
